# Initial kernel scaffold; baseline (speedup 1.0000x reference)
import jax
import jax.numpy as jnp
from jax.experimental import pallas as pl


def kernel(z, quantized_z, codebook):
    raise NotImplementedError("write your pallas kernel here")


# speedup vs baseline: 2.4482x; 2.4482x over previous
import functools

import jax
import jax.numpy as jnp
from jax import lax
from jax.experimental import pallas as pl
from jax.experimental.pallas import tpu as pltpu
from jax.experimental.pallas import tpu_sc as plsc

NC = 2
NS = 16
NW = NC * NS
L = 16

N_TOTAL = 256 * 1024 * 64
PER_TILE = N_TOTAL // NW
CH = 16384
CPT = PER_TILE // CH
UNROLL = 8
NVI = CH // (L * UNROLL)

CB_TOTAL = 8192 * 64
CB_PER_TILE = CB_TOTAL // NW

_mesh = plsc.VectorSubcoreMesh(core_axis_name="c", subcore_axis_name="s")


@functools.partial(
    pl.kernel,
    out_type=jax.ShapeDtypeStruct((NW, L), jnp.float32),
    mesh=_mesh,
    compiler_params=pltpu.CompilerParams(needs_layout_passes=False),
    scratch_types=[
        pltpu.VMEM((CH,), jnp.float32),
        pltpu.VMEM((CH,), jnp.float32),
        pltpu.VMEM((CH,), jnp.float32),
        pltpu.VMEM((CH,), jnp.float32),
        pltpu.VMEM((L,), jnp.float32),
        pltpu.SemaphoreType.DMA,
        pltpu.SemaphoreType.DMA,
    ],
)
def _sc_stats(z_hbm, q_hbm, cb_hbm, out_hbm, zb0, zb1, qb0, qb1, ob, sem0, sem1):
    wid = lax.axis_index("s") * NC + lax.axis_index("c")
    zbufs = (zb0, zb1)
    qbufs = (qb0, qb1)
    sems = (sem0, sem1)

    def start(slot, c):
        dz = pltpu.async_copy(z_hbm.at[wid, c], zbufs[slot], sems[slot])
        dq = pltpu.async_copy(q_hbm.at[wid, c], qbufs[slot], sems[slot])
        return (dz, dq)

    neg_inf = jnp.full((L,), -jnp.inf, jnp.float32)
    pos_inf = jnp.full((L,), jnp.inf, jnp.float32)
    zero = jnp.zeros((L,), jnp.float32)

    accs = [zero] * UNROLL
    vmn = pos_inf
    vmx = neg_inf

    pend = [None, None]
    pend[0] = start(0, 0)
    for c in range(CPT):
        slot = c & 1
        if c + 1 < CPT:
            pend[1 - slot] = start(1 - slot, c + 1)
        dz, dq = pend[slot]
        dz.wait()
        dq.wait()
        zb = zbufs[slot]
        qb = qbufs[slot]

        def body(i, carry):
            a = list(carry[:UNROLL])
            mn = carry[UNROLL]
            mx = carry[UNROLL + 1]
            base = i * (L * UNROLL)
            for u in range(UNROLL):
                x = zb[pl.ds(base + u * L, L)]
                y = qb[pl.ds(base + u * L, L)]
                d = x - y
                a[u] = a[u] + d * d
                mn = jnp.minimum(mn, y)
                mx = jnp.maximum(mx, y)
            return tuple(a) + (mn, mx)

        carry = lax.fori_loop(0, NVI, body, tuple(accs) + (vmn, vmx))
        accs = list(carry[:UNROLL])
        vmn = carry[UNROLL]
        vmx = carry[UNROLL + 1]

    pltpu.sync_copy(cb_hbm.at[wid], zb0)
    caccs = [zero] * UNROLL

    def cb_body(i, carry):
        a = list(carry)
        base = i * (L * UNROLL)
        for u in range(UNROLL):
            x = zb0[pl.ds(base + u * L, L)]
            a[u] = a[u] + x * x
        return tuple(a)

    caccs = list(lax.fori_loop(0, NVI, cb_body, tuple(caccs)))

    acc = accs[0]
    cacc = caccs[0]
    for u in range(1, UNROLL):
        acc = acc + accs[u]
        cacc = cacc + caccs[u]

    s_sum = jnp.sum(acc)
    s_mn = jnp.min(vmn)
    s_mx = jnp.max(vmx)
    s_cb = jnp.sum(cacc)

    lanes = lax.iota(jnp.int32, L)
    row = jnp.where(
        lanes == 0,
        s_sum,
        jnp.where(lanes == 1, s_mn, jnp.where(lanes == 2, s_mx, jnp.where(lanes == 3, s_cb, 0.0))),
    )
    ob[...] = row
    pltpu.sync_copy(ob, out_hbm.at[wid])


NBINS = 100


@functools.partial(
    pl.kernel,
    out_type=jax.ShapeDtypeStruct((NW, NBINS, L), jnp.float32),
    mesh=_mesh,
    compiler_params=pltpu.CompilerParams(needs_layout_passes=False),
    scratch_types=[
        pltpu.VMEM((CH,), jnp.float32),
        pltpu.VMEM((CH,), jnp.float32),
        pltpu.VMEM((NW, L), jnp.float32),
        pltpu.VMEM((NBINS, L), jnp.float32),
        pltpu.SemaphoreType.DMA,
        pltpu.SemaphoreType.DMA,
    ],
)
def _sc_hist(q_hbm, stats_hbm, out_hbm, qb0, qb1, sb, hb, sem0, sem1):
    wid = lax.axis_index("s") * NC + lax.axis_index("c")
    qbufs = (qb0, qb1)
    sems = (sem0, sem1)

    pltpu.sync_copy(stats_hbm, sb)

    def mnmx_body(i, carry):
        mn_v, mx_v = carry
        row = sb[i, :]
        return (jnp.minimum(mn_v, row), jnp.maximum(mx_v, row))

    mn_v, mx_v = lax.fori_loop(
        0,
        NW,
        mnmx_body,
        (jnp.full((L,), jnp.inf, jnp.float32), jnp.full((L,), -jnp.inf, jnp.float32)),
    )
    mnv = jnp.full((L,), mn_v[1], jnp.float32)
    mxv = jnp.full((L,), mx_v[2], jnp.float32)
    widthv = jnp.where(mxv > mnv, mxv - mnv, jnp.full((L,), 1.0, jnp.float32))
    rsv = jnp.full((L,), float(NBINS), jnp.float32) / widthv
    ones = jnp.ones((L,), jnp.float32)
    lanes = lax.iota(jnp.int32, L)
    cmax = jnp.full((L,), NBINS - 1, jnp.int32)
    czero = jnp.zeros((L,), jnp.int32)

    def zero_body(i, _):
        hb[i, :] = jnp.zeros((L,), jnp.float32)
        return 0

    lax.fori_loop(0, NBINS, zero_body, 0)

    def start(slot, c):
        return pltpu.async_copy(q_hbm.at[wid, c], qbufs[slot], sems[slot])

    pend = [None, None]
    pend[0] = start(0, 0)
    for c in range(CPT):
        slot = c & 1
        if c + 1 < CPT:
            pend[1 - slot] = start(1 - slot, c + 1)
        pend[slot].wait()
        qb = qbufs[slot]

        def body(i, carry):
            base = i * (L * UNROLL)
            for u in range(UNROLL):
                y = qb[pl.ds(base + u * L, L)]
                t = (y - mnv) * rsv
                idx = t.astype(jnp.int32)
                idx = jnp.minimum(idx, cmax)
                idx = jnp.maximum(idx, czero)
                plsc.addupdate_scatter(hb, [idx, lanes], ones)
            return 0

        lax.fori_loop(0, NVI, body, 0)

    pltpu.sync_copy(hb, out_hbm.at[wid])


def _combine_body(stats_ref, histp_ref, out_ref):
    stats = stats_ref[...]
    hp = histp_ref[...]
    s_sum = jnp.sum(stats[:, 0])
    s_cb = jnp.sum(stats[:, 3])
    hist = jnp.sum(jnp.sum(hp, axis=0), axis=1)
    n = jnp.float32(N_TOTAL)
    mse = s_sum / n
    probs = hist / jnp.sum(hist)
    ent = -jnp.sum(probs * jnp.log(probs + 1e-10))
    l2 = jnp.sqrt(s_cb)
    total = mse + 0.25 * mse - 0.1 * ent + 0.01 * l2
    lanes = lax.iota(jnp.int32, 8)
    out_ref[...] = jnp.where(
        lanes == 0,
        total,
        jnp.where(
            lanes == 1,
            mse,
            jnp.where(lanes == 2, mse, jnp.where(lanes == 3, ent, jnp.where(lanes == 4, l2, 0.0))),
        ),
    )


def kernel(z, quantized_z, codebook):
    z2 = z.reshape(NW, CPT, CH)
    q2 = quantized_z.reshape(NW, CPT, CH)
    cb2 = codebook.reshape(NW, CB_PER_TILE)

    stats = _sc_stats(z2, q2, cb2)
    histp = _sc_hist(q2, stats)

    out8 = pl.pallas_call(
        _combine_body,
        out_shape=jax.ShapeDtypeStruct((8,), jnp.float32),
    )(stats, histp)

    return (out8[0], out8[1], out8[2], out8[3], out8[4])
